# Initial kernel scaffold; baseline (speedup 1.0000x reference)
import jax
import jax.numpy as jnp
from jax.experimental import pallas as pl


def kernel(pc, feats, W0, b0, Wl0, bl0, Ws0, bs0, Wf0, bf0, Wl1, bl1, Ws1, bs1, Wf1, bf1, W1, b1, Wr, br):
    raise NotImplementedError("write your pallas kernel here")


# speedup vs baseline: 12.8937x; 12.8937x over previous
import functools

import jax
import jax.numpy as jnp
from jax import lax
from jax.experimental import pallas as pl
from jax.experimental.pallas import tpu as pltpu
from jax.experimental.pallas import tpu_sc as plsc

K = 16
TBLD = 128


def _stage_a_body(feats_ref, pc_ref, w0_ref, b0_ref, wr_ref, br_ref,
                  tbl_ref, res_ref):
    f = feats_ref[...]
    y0 = jnp.maximum(jnp.dot(f, w0_ref[...],
                             preferred_element_type=jnp.float32)
                     + b0_ref[...], 0.0)
    tbl_ref[:, 0:64] = y0
    tbl_ref[:, 64:66] = pc_ref[...]
    tbl_ref[:, 66:TBLD] = jnp.zeros((y0.shape[0], TBLD - 66), jnp.float32)
    res_ref[...] = jnp.maximum(jnp.dot(f, wr_ref[...],
                                       preferred_element_type=jnp.float32)
                               + br_ref[...], 0.0)


def _stage_a(featsf, pcf, W0, b0r, Wr, brr):
    BN, IU = featsf.shape
    T = 512
    grid = (BN // T,)
    return pl.pallas_call(
        _stage_a_body,
        grid=grid,
        in_specs=[
            pl.BlockSpec((T, IU), lambda i: (i, 0)),
            pl.BlockSpec((T, 2), lambda i: (i, 0)),
            pl.BlockSpec(W0.shape, lambda i: (0, 0)),
            pl.BlockSpec(b0r.shape, lambda i: (0, 0)),
            pl.BlockSpec(Wr.shape, lambda i: (0, 0)),
            pl.BlockSpec(brr.shape, lambda i: (0, 0)),
        ],
        out_specs=[
            pl.BlockSpec((T, TBLD), lambda i: (i, 0)),
            pl.BlockSpec((T, Wr.shape[1]), lambda i: (i, 0)),
        ],
        out_shape=[
            jax.ShapeDtypeStruct((BN, TBLD), jnp.float32),
            jax.ShapeDtypeStruct((BN, Wr.shape[1]), jnp.float32),
        ],
    )(featsf, pcf, W0, b0r, Wr, brr)


def _stage_b_body(pc_ref, pct_ref, idx_ref, *, n):
    b = pl.program_id(0)
    pct = pct_ref[...][0]
    xs = pct[0:1, :]
    ys = pct[1:2, :]
    xi = pc_ref[:, 0:1]
    yi = pc_ref[:, 1:2]
    dx = xi - xs
    dy = yi - ys
    d = dx * dx + dy * dy
    lanes = lax.broadcasted_iota(jnp.int32, d.shape, 1)
    for k in range(K):
        mn = jnp.min(d, axis=1, keepdims=True)
        cand = jnp.where(d == mn, lanes, n)
        j = jnp.min(cand, axis=1, keepdims=True)
        idx_ref[:, k:k + 1] = j + b * n
        d = jnp.where(lanes == j, jnp.float32(jnp.inf), d)


def _stage_b(pcf, pcTp, Bb, N):
    T = 256
    NT = N // T
    grid = (Bb, NT)
    return pl.pallas_call(
        functools.partial(_stage_b_body, n=N),
        grid=grid,
        in_specs=[
            pl.BlockSpec((T, 2), lambda b, t: (b * NT + t, 0)),
            pl.BlockSpec((1, 8, N), lambda b, t: (b, 0, 0)),
        ],
        out_specs=pl.BlockSpec((T, K), lambda b, t: (b * NT + t, 0)),
        out_shape=jax.ShapeDtypeStruct((Bb * N, K), jnp.int32),
    )(pcf, pcTp)


def _sc_gather(table, idxf, D):
    M = idxf.shape[0]
    info = plsc.get_sparse_core_info()
    NW = info.num_cores * info.num_subcores
    b_per_w = M // NW
    CH = 512
    n_ch = b_per_w // CH
    mesh = plsc.VectorSubcoreMesh(core_axis_name="c", subcore_axis_name="s")

    @functools.partial(
        pl.kernel,
        mesh=mesh,
        out_type=jax.ShapeDtypeStruct((M, D), jnp.float32),
        scratch_types=[
            pltpu.VMEM((CH,), jnp.int32),
            pltpu.VMEM((CH, D), jnp.float32),
            pltpu.SemaphoreType.DMA,
        ],
    )
    def gk(table_hbm, idx_hbm, out_hbm, idx_v, rows_v, sem):
        wid = lax.axis_index("s") * info.num_cores + lax.axis_index("c")
        base = wid * b_per_w
        for c in range(n_ch):
            off = base + c * CH
            pltpu.sync_copy(idx_hbm.at[pl.ds(off, CH)], idx_v)
            pltpu.async_copy(table_hbm.at[idx_v], rows_v, sem).wait()
            pltpu.sync_copy(rows_v, out_hbm.at[pl.ds(off, CH)])

    return gk(table, idxf)


def _locse(g_coords_x, g_coords_y, px, py, w_ref, b_ref):
    rx = px - g_coords_x
    ry = py - g_coords_y
    nrm = jnp.sqrt(rx * rx + ry * ry + 1e-12)
    w = w_ref[...]

    def row(c):
        return w[c:c + 1, :][None]

    lin = (px * row(0) + py * row(1)
           + g_coords_x * row(2) + g_coords_y * row(3)
           + rx * row(4) + ry * row(5) + nrm * row(6)
           + b_ref[...][None])
    return jnp.maximum(lin, 0.0)


def _att_pool(nf, ws_ref, bs_ref, wf_ref, bf_ref):
    T, Kk, C = nf.shape
    s = (jnp.dot(nf.reshape(T * Kk, C), ws_ref[...],
                 preferred_element_type=jnp.float32)
         + bs_ref[...]).reshape(T, Kk, C)
    mx = jnp.max(s, axis=-1, keepdims=True)
    e = jnp.exp(s - mx)
    sm = e / jnp.sum(e, axis=-1, keepdims=True)
    pooled = jnp.sum(nf * sm, axis=1)
    return jnp.maximum(jnp.dot(pooled, wf_ref[...],
                               preferred_element_type=jnp.float32)
                       + bf_ref[...], 0.0)


def _stage_c_body(g0_ref, pc_ref, wl0_ref, bl0_ref, ws0_ref, bs0_ref,
                  wf0_ref, bf0_ref, wl1_ref, bl1_ref, y1_ref, r1_ref, *, t):
    g0 = g0_ref[...]
    gy = g0[:, 0:64].reshape(t, K, 64)
    npx = g0[:, 64:65].reshape(t, K, 1)
    npy = g0[:, 65:66].reshape(t, K, 1)
    pc_t = pc_ref[...]
    px = pc_t[:, 0:1][:, None, :]
    py = pc_t[:, 1:2][:, None, :]
    r0 = _locse(npx, npy, px, py, wl0_ref, bl0_ref)
    nf = jnp.concatenate([gy, r0], axis=-1)
    y1_ref[:, 0:64] = _att_pool(nf, ws0_ref, bs0_ref, wf0_ref, bf0_ref)
    y1_ref[:, 64:128] = jnp.zeros((t, 64), jnp.float32)
    r1 = _locse(npx, npy, px, py, wl1_ref, bl1_ref)
    r1_ref[...] = r1.reshape(t * K, 64)


def _stage_c(g0, pcf, Wl0p, bl0r, Ws0, bs0r, Wf0, bf0r, Wl1p, bl1r):
    BN = pcf.shape[0]
    T = 128
    grid = (BN // T,)
    wspec = lambda a: pl.BlockSpec(a.shape, lambda i: (0, 0))
    return pl.pallas_call(
        functools.partial(_stage_c_body, t=T),
        grid=grid,
        in_specs=[
            pl.BlockSpec((T * K, TBLD), lambda i: (i, 0)),
            pl.BlockSpec((T, 2), lambda i: (i, 0)),
            wspec(Wl0p), wspec(bl0r), wspec(Ws0), wspec(bs0r),
            wspec(Wf0), wspec(bf0r), wspec(Wl1p), wspec(bl1r),
        ],
        out_specs=[
            pl.BlockSpec((T, 128), lambda i: (i, 0)),
            pl.BlockSpec((T * K, 64), lambda i: (i, 0)),
        ],
        out_shape=[
            jax.ShapeDtypeStruct((BN, 128), jnp.float32),
            jax.ShapeDtypeStruct((BN * K, 64), jnp.float32),
        ],
    )(g0, pcf, Wl0p, bl0r, Ws0, bs0r, Wf0, bf0r, Wl1p, bl1r)


def _stage_d_body(g1_ref, r1_ref, res_ref, ws1_ref, bs1_ref, wf1_ref,
                  bf1_ref, w1_ref, b1_ref, out_ref, *, t):
    g1 = g1_ref[:, 0:64].reshape(t, K, 64)
    r1 = r1_ref[...].reshape(t, K, 64)
    nf = jnp.concatenate([g1, r1], axis=-1)
    a1 = _att_pool(nf, ws1_ref, bs1_ref, wf1_ref, bf1_ref)
    y = jnp.maximum(jnp.dot(a1, w1_ref[...],
                            preferred_element_type=jnp.float32)
                    + b1_ref[...], 0.0)
    out_ref[...] = jnp.maximum(y + res_ref[...], 0.0)


def _stage_d(g1, r1, res, Ws1, bs1r, Wf1, bf1r, W1, b1r):
    BN = res.shape[0]
    U = W1.shape[1]
    T = 128
    grid = (BN // T,)
    wspec = lambda a: pl.BlockSpec(a.shape, lambda i: (0, 0))
    return pl.pallas_call(
        functools.partial(_stage_d_body, t=T),
        grid=grid,
        in_specs=[
            pl.BlockSpec((T * K, 128), lambda i: (i, 0)),
            pl.BlockSpec((T * K, 64), lambda i: (i, 0)),
            pl.BlockSpec((T, U), lambda i: (i, 0)),
            wspec(Ws1), wspec(bs1r), wspec(Wf1), wspec(bf1r),
            wspec(W1), wspec(b1r),
        ],
        out_specs=pl.BlockSpec((T, U), lambda i: (i, 0)),
        out_shape=jax.ShapeDtypeStruct((BN, U), jnp.float32),
    )(g1, r1, res, Ws1, bs1r, Wf1, bf1r, W1, b1r)


def kernel(pc, feats, W0, b0, Wl0, bl0, Ws0, bs0, Wf0, bf0,
           Wl1, bl1, Ws1, bs1, Wf1, bf1, W1, b1, Wr, br):
    Bb, N, _ = pc.shape
    BN = Bb * N
    pcf = pc.reshape(BN, 2)
    featsf = feats.reshape(BN, feats.shape[-1])
    pcTp = jnp.pad(pc.transpose(0, 2, 1), ((0, 0), (0, 6), (0, 0)))
    Wl0p = jnp.pad(Wl0, ((0, 1), (0, 0)))
    Wl1p = jnp.pad(Wl1, ((0, 1), (0, 0)))
    _r = lambda v: v.reshape(1, -1)

    tbl, res = _stage_a(featsf, pcf, W0, _r(b0), Wr, _r(br))
    idxf = _stage_b(pcf, pcTp, Bb, N).reshape(BN * K)
    g0 = _sc_gather(tbl, idxf, TBLD)
    y1, r1m = _stage_c(g0, pcf, Wl0p, _r(bl0), Ws0, _r(bs0),
                       Wf0, _r(bf0), Wl1p, _r(bl1))
    g1 = _sc_gather(y1, idxf, 128)
    out = _stage_d(g1, r1m, res, Ws1, _r(bs1), Wf1, _r(bf1), W1, _r(b1))
    return out.reshape(Bb, N, W1.shape[1])


# speedup vs baseline: 13.5621x; 1.0518x over previous
import functools

import jax
import jax.numpy as jnp
from jax import lax
from jax.experimental import pallas as pl
from jax.experimental.pallas import tpu as pltpu
from jax.experimental.pallas import tpu_sc as plsc

K = 16
TBLD = 128


def _stage_a_body(feats_ref, pc_ref, w0_ref, b0_ref, wr_ref, br_ref,
                  tbl_ref, res_ref):
    f = feats_ref[...]
    y0 = jnp.maximum(jnp.dot(f, w0_ref[...],
                             preferred_element_type=jnp.float32)
                     + b0_ref[...], 0.0)
    tbl_ref[:, 0:64] = y0
    tbl_ref[:, 64:66] = pc_ref[...]
    tbl_ref[:, 66:TBLD] = jnp.zeros((y0.shape[0], TBLD - 66), jnp.float32)
    res_ref[...] = jnp.maximum(jnp.dot(f, wr_ref[...],
                                       preferred_element_type=jnp.float32)
                               + br_ref[...], 0.0)


def _stage_a(featsf, pcf, W0, b0r, Wr, brr):
    BN, IU = featsf.shape
    T = 512
    grid = (BN // T,)
    return pl.pallas_call(
        _stage_a_body,
        grid=grid,
        in_specs=[
            pl.BlockSpec((T, IU), lambda i: (i, 0)),
            pl.BlockSpec((T, 2), lambda i: (i, 0)),
            pl.BlockSpec(W0.shape, lambda i: (0, 0)),
            pl.BlockSpec(b0r.shape, lambda i: (0, 0)),
            pl.BlockSpec(Wr.shape, lambda i: (0, 0)),
            pl.BlockSpec(brr.shape, lambda i: (0, 0)),
        ],
        out_specs=[
            pl.BlockSpec((T, TBLD), lambda i: (i, 0)),
            pl.BlockSpec((T, Wr.shape[1]), lambda i: (i, 0)),
        ],
        out_shape=[
            jax.ShapeDtypeStruct((BN, TBLD), jnp.float32),
            jax.ShapeDtypeStruct((BN, Wr.shape[1]), jnp.float32),
        ],
    )(featsf, pcf, W0, b0r, Wr, brr)


def _stage_b_body(pc_ref, pct_ref, idx_ref, d16_ref, *, n):
    b = pl.program_id(0)
    pct = pct_ref[...][0]
    xs = pct[0:1, :]
    ys = pct[1:2, :]
    xi = pc_ref[:, 0:1]
    yi = pc_ref[:, 1:2]
    dx = xi - xs
    dy = yi - ys
    d = dx * dx + dy * dy
    lanes = lax.broadcasted_iota(jnp.int32, d.shape, 1)
    bits = lax.bitcast_convert_type(d, jnp.int32)
    key = lax.bitcast_convert_type((bits & ~(n - 1)) | lanes, jnp.float32)
    inf = jnp.float32(jnp.inf)
    for k in range(K):
        mn = jnp.min(key, axis=1, keepdims=True)
        mbits = lax.bitcast_convert_type(mn, jnp.int32)
        idx_ref[:, k:k + 1] = (mbits & (n - 1)) + b * n
        d16_ref[:, k:k + 1] = lax.bitcast_convert_type(
            mbits & ~(n - 1), jnp.float32)
        key = jnp.where(key == mn, inf, key)


def _stage_b(pcf, pcTp, Bb, N):
    T = 256
    NT = N // T
    grid = (Bb, NT)
    return pl.pallas_call(
        functools.partial(_stage_b_body, n=N),
        grid=grid,
        in_specs=[
            pl.BlockSpec((T, 2), lambda b, t: (b * NT + t, 0)),
            pl.BlockSpec((1, 8, N), lambda b, t: (b, 0, 0)),
        ],
        out_specs=[
            pl.BlockSpec((T, K), lambda b, t: (b * NT + t, 0)),
            pl.BlockSpec((T, K), lambda b, t: (b * NT + t, 0)),
        ],
        out_shape=[
            jax.ShapeDtypeStruct((Bb * N, K), jnp.int32),
            jax.ShapeDtypeStruct((Bb * N, K), jnp.float32),
        ],
    )(pcf, pcTp)


def _sc_gather(table, idxf, D):
    M = idxf.shape[0]
    info = plsc.get_sparse_core_info()
    NW = info.num_cores * info.num_subcores
    b_per_w = M // NW
    CH = 512
    n_ch = b_per_w // CH
    mesh = plsc.VectorSubcoreMesh(core_axis_name="c", subcore_axis_name="s")

    @functools.partial(
        pl.kernel,
        mesh=mesh,
        out_type=jax.ShapeDtypeStruct((M, D), jnp.float32),
        scratch_types=[
            pltpu.VMEM((CH,), jnp.int32),
            pltpu.VMEM((CH, D), jnp.float32),
            pltpu.SemaphoreType.DMA,
        ],
    )
    def gk(table_hbm, idx_hbm, out_hbm, idx_v, rows_v, sem):
        wid = lax.axis_index("s") * info.num_cores + lax.axis_index("c")
        base = wid * b_per_w
        for c in range(n_ch):
            off = base + c * CH
            pltpu.sync_copy(idx_hbm.at[pl.ds(off, CH)], idx_v)
            pltpu.async_copy(table_hbm.at[idx_v], rows_v, sem).wait()
            pltpu.sync_copy(rows_v, out_hbm.at[pl.ds(off, CH)])

    return gk(table, idxf)


def _locse_lin(pxy, npxy, nrmcol, w_ref, b_ref, t):
    w = w_ref[...]
    ci = jnp.dot(pxy, w[0:2] + w[4:6],
                 preferred_element_type=jnp.float32)
    cj = jnp.dot(npxy, w[2:4] - w[4:6],
                 preferred_element_type=jnp.float32)
    cib = jnp.broadcast_to(ci[:, None, :], (t, K, 64)).reshape(t * K, 64)
    lin = cib + cj + nrmcol * w[6:7] + b_ref[...]
    return jnp.maximum(lin, 0.0)


def _att_pool(nf, ws_ref, bs_ref, wf_ref, bf_ref):
    T, Kk, C = nf.shape
    s = (jnp.dot(nf.reshape(T * Kk, C), ws_ref[...],
                 preferred_element_type=jnp.float32)
         + bs_ref[...]).reshape(T, Kk, C)
    e = jnp.exp(s)
    sm = e / jnp.sum(e, axis=-1, keepdims=True)
    pooled = jnp.sum(nf * sm, axis=1)
    return jnp.maximum(jnp.dot(pooled, wf_ref[...],
                               preferred_element_type=jnp.float32)
                       + bf_ref[...], 0.0)


def _stage_c_body(g0_ref, pc_ref, d16_ref, wl0_ref, bl0_ref, ws0_ref,
                  bs0_ref, wf0_ref, bf0_ref, wl1_ref, bl1_ref,
                  y1_ref, r1_ref, *, t):
    g0 = g0_ref[...]
    gy = g0[:, 0:64].reshape(t, K, 64)
    npxy = g0[:, 64:66]
    pxy = pc_ref[...]
    nrmcol = jnp.sqrt(d16_ref[...] + 1e-12)
    r0 = _locse_lin(pxy, npxy, nrmcol, wl0_ref, bl0_ref, t)
    nf = jnp.concatenate([gy, r0.reshape(t, K, 64)], axis=-1)
    y1_ref[:, 0:64] = _att_pool(nf, ws0_ref, bs0_ref, wf0_ref, bf0_ref)
    y1_ref[:, 64:128] = jnp.zeros((t, 64), jnp.float32)
    r1_ref[...] = _locse_lin(pxy, npxy, nrmcol, wl1_ref, bl1_ref, t)


def _stage_c(g0, pcf, d16col, Wl0p, bl0r, Ws0, bs0r, Wf0, bf0r, Wl1p, bl1r):
    BN = pcf.shape[0]
    T = 256
    grid = (BN // T,)
    wspec = lambda a: pl.BlockSpec(a.shape, lambda i: (0, 0))
    return pl.pallas_call(
        functools.partial(_stage_c_body, t=T),
        grid=grid,
        in_specs=[
            pl.BlockSpec((T * K, TBLD), lambda i: (i, 0)),
            pl.BlockSpec((T, 2), lambda i: (i, 0)),
            pl.BlockSpec((T * K, 1), lambda i: (i, 0)),
            wspec(Wl0p), wspec(bl0r), wspec(Ws0), wspec(bs0r),
            wspec(Wf0), wspec(bf0r), wspec(Wl1p), wspec(bl1r),
        ],
        out_specs=[
            pl.BlockSpec((T, 128), lambda i: (i, 0)),
            pl.BlockSpec((T * K, 64), lambda i: (i, 0)),
        ],
        out_shape=[
            jax.ShapeDtypeStruct((BN, 128), jnp.float32),
            jax.ShapeDtypeStruct((BN * K, 64), jnp.float32),
        ],
    )(g0, pcf, d16col, Wl0p, bl0r, Ws0, bs0r, Wf0, bf0r, Wl1p, bl1r)


def _stage_d_body(g1_ref, r1_ref, res_ref, ws1_ref, bs1_ref, wf1_ref,
                  bf1_ref, w1_ref, b1_ref, out_ref, *, t):
    g1 = g1_ref[:, 0:64].reshape(t, K, 64)
    r1 = r1_ref[...].reshape(t, K, 64)
    nf = jnp.concatenate([g1, r1], axis=-1)
    a1 = _att_pool(nf, ws1_ref, bs1_ref, wf1_ref, bf1_ref)
    y = jnp.maximum(jnp.dot(a1, w1_ref[...],
                            preferred_element_type=jnp.float32)
                    + b1_ref[...], 0.0)
    out_ref[...] = jnp.maximum(y + res_ref[...], 0.0)


def _stage_d(g1, r1, res, Ws1, bs1r, Wf1, bf1r, W1, b1r):
    BN = res.shape[0]
    U = W1.shape[1]
    T = 256
    grid = (BN // T,)
    wspec = lambda a: pl.BlockSpec(a.shape, lambda i: (0, 0))
    return pl.pallas_call(
        functools.partial(_stage_d_body, t=T),
        grid=grid,
        in_specs=[
            pl.BlockSpec((T * K, 128), lambda i: (i, 0)),
            pl.BlockSpec((T * K, 64), lambda i: (i, 0)),
            pl.BlockSpec((T, U), lambda i: (i, 0)),
            wspec(Ws1), wspec(bs1r), wspec(Wf1), wspec(bf1r),
            wspec(W1), wspec(b1r),
        ],
        out_specs=pl.BlockSpec((T, U), lambda i: (i, 0)),
        out_shape=jax.ShapeDtypeStruct((BN, U), jnp.float32),
    )(g1, r1, res, Ws1, bs1r, Wf1, bf1r, W1, b1r)


def kernel(pc, feats, W0, b0, Wl0, bl0, Ws0, bs0, Wf0, bf0,
           Wl1, bl1, Ws1, bs1, Wf1, bf1, W1, b1, Wr, br):
    Bb, N, _ = pc.shape
    BN = Bb * N
    pcf = pc.reshape(BN, 2)
    featsf = feats.reshape(BN, feats.shape[-1])
    pcTp = jnp.pad(pc.transpose(0, 2, 1), ((0, 0), (0, 6), (0, 0)))
    Wl0p = jnp.pad(Wl0, ((0, 1), (0, 0)))
    Wl1p = jnp.pad(Wl1, ((0, 1), (0, 0)))
    _r = lambda v: v.reshape(1, -1)

    tbl, res = _stage_a(featsf, pcf, W0, _r(b0), Wr, _r(br))
    idx2, d16 = _stage_b(pcf, pcTp, Bb, N)
    idxf = idx2.reshape(BN * K)
    g0 = _sc_gather(tbl, idxf, TBLD)
    y1, r1m = _stage_c(g0, pcf, d16.reshape(BN * K, 1), Wl0p, _r(bl0),
                       Ws0, _r(bs0), Wf0, _r(bf0), Wl1p, _r(bl1))
    g1 = _sc_gather(y1, idxf, 128)
    out = _stage_d(g1, r1m, res, Ws1, _r(bs1), Wf1, _r(bf1), W1, _r(b1))
    return out.reshape(Bb, N, W1.shape[1])
